# baseline (speedup 1.0000x reference)
import functools

import jax
import jax.numpy as jnp
from jax import lax
from jax.experimental import pallas as pl
from jax.experimental.pallas import tpu as pltpu
from jax.experimental.pallas import tpu_sc as plsc

_N = 10000
_E = 320000
_H = 128
_G = 64
_C = 3

_NC = 2
_NS = 16
_NW = _NC * _NS
_EPW = _E // _NW
_CH = 125
_NCH = _EPW // _CH
_NBUF = 2
_PH = 40
_TPW = 10240 // _NS
_DW = 128
_NP = 10240
_PAD = _NP - _N

_mesh = plsc.VectorSubcoreMesh(core_axis_name="c", subcore_axis_name="s")


@functools.partial(
    pl.kernel,
    out_type=jax.ShapeDtypeStruct((_NW * _NP,), jnp.float32),
    mesh=_mesh,
    compiler_params=pltpu.CompilerParams(needs_layout_passes=False),
    scratch_types=[
        pltpu.VMEM((_EPW,), jnp.int32),
        pltpu.VMEM((_NP,), jnp.float32),
    ],
)
def _sc_degree(dst_flat, out, dstv, hist):
    c = lax.axis_index("c")
    s = lax.axis_index("s")
    w = s * _NC + c
    pltpu.sync_copy(dst_flat.at[pl.ds(w * _EPW, _EPW)], dstv)
    zero16 = jnp.zeros((16,), jnp.float32)

    def zbody(i, carry):
        hist[pl.ds(i * 16, 16)] = zero16
        return carry

    lax.fori_loop(0, _NP // 16, zbody, 0)
    ones16 = jnp.ones((16,), jnp.float32)

    def body(i, carry):
        for u in range(5):
            idx = dstv[pl.ds((i * 5 + u) * 16, 16)]
            plsc.addupdate_scatter(hist, [idx], ones16)
        return carry

    lax.fori_loop(0, _EPW // 80, body, 0)
    pltpu.sync_copy(hist, out.at[pl.ds(w * _NP, _NP)])


@functools.partial(
    pl.kernel,
    out_type=jax.ShapeDtypeStruct((_NC, _NP, _H), jnp.float32),
    mesh=_mesh,
    scratch_types=[
        pltpu.VMEM((_PH, _CH), jnp.int32),
        pltpu.VMEM((_PH, _CH), jnp.int32),
    ] + [pltpu.VMEM((_CH, _H), jnp.float32) for _ in range(_NBUF)]
      + [pltpu.SemaphoreType.DMA for _ in range(2 * _NBUF)]
      + [pltpu.SemaphoreType.DMA for _ in range(2)]
      + [pltpu.VMEM_SHARED((_NP, _H), jnp.float32)],
)
def _sc_edge_sum(g_hbm, src_r, dst_r, zero_hbm, out, srcv, dstv, *rest):
    c = lax.axis_index("c")
    s = lax.axis_index("s")
    bufs = rest[:_NBUF]
    gsems = rest[_NBUF:3 * _NBUF]
    ssems = rest[3 * _NBUF:3 * _NBUF + 2]
    acc = rest[-1]
    w = s * _NC + c
    _H0 = 64
    _H1 = _CH - _H0

    def _gather(j, b):
        pltpu.async_copy(g_hbm.at[srcv.at[j, pl.ds(0, _H0)]],
                         bufs[b].at[pl.ds(0, _H0)], gsems[2 * b])
        pltpu.async_copy(g_hbm.at[srcv.at[j, pl.ds(_H0, _H1)]],
                         bufs[b].at[pl.ds(_H0, _H1)], gsems[2 * b + 1])

    def _gwait(j, b):
        pltpu.make_async_copy(g_hbm.at[srcv.at[j, pl.ds(0, _H0)]],
                              bufs[b].at[pl.ds(0, _H0)], gsems[2 * b]).wait()
        pltpu.make_async_copy(g_hbm.at[srcv.at[j, pl.ds(_H0, _H1)]],
                              bufs[b].at[pl.ds(_H0, _H1)],
                              gsems[2 * b + 1]).wait()

    def _scatter(j, b):
        pltpu.async_copy(bufs[b].at[pl.ds(0, _H0)],
                         acc.at[dstv.at[j, pl.ds(0, _H0)]], ssems[0],
                         add=True)
        pltpu.async_copy(bufs[b].at[pl.ds(_H0, _H1)],
                         acc.at[dstv.at[j, pl.ds(_H0, _H1)]], ssems[1],
                         add=True)
        pltpu.make_async_copy(bufs[b].at[pl.ds(0, _H0)],
                              acc.at[dstv.at[j, pl.ds(0, _H0)]],
                              ssems[0]).wait()
        pltpu.make_async_copy(bufs[b].at[pl.ds(_H0, _H1)],
                              acc.at[dstv.at[j, pl.ds(_H0, _H1)]],
                              ssems[1]).wait()

    for ph in range(_NCH // _PH):
        pltpu.sync_copy(src_r.at[w, pl.ds(ph * _PH, _PH)], srcv)
        pltpu.sync_copy(dst_r.at[w, pl.ds(ph * _PH, _PH)], dstv)
        for b in range(_NBUF):
            _gather(b, b)
        if ph == 0:
            @pl.when((c == 0) & (s < _NS - 1))
            def _():
                pltpu.sync_copy(g_hbm.at[pl.ds(s * _TPW, _TPW)],
                                acc.at[pl.ds(s * _TPW, _TPW)])

            @pl.when((c == 0) & (s == _NS - 1))
            def _():
                pltpu.sync_copy(g_hbm.at[pl.ds(_N - _TPW + _PAD, _TPW - _PAD)],
                                acc.at[pl.ds(_N - _TPW + _PAD, _TPW - _PAD)])
                pltpu.sync_copy(zero_hbm.at[pl.ds(0, _PAD)],
                                acc.at[pl.ds(_N, _PAD)])

            @pl.when(c == 1)
            def _():
                pltpu.sync_copy(zero_hbm, acc.at[pl.ds(s * _TPW, _TPW)])

            plsc.subcore_barrier()

        def body(i, carry):
            j0 = i * _NBUF
            for b in range(_NBUF):
                j = j0 + b
                _gwait(j, b)
                _scatter(j, b)

                @pl.when(j + _NBUF < _PH)
                def _():
                    _gather(j + _NBUF, b)
            return carry

        lax.fori_loop(0, _PH // _NBUF, body, 0)
    plsc.subcore_barrier()
    pltpu.sync_copy(acc.at[pl.ds(s * _TPW, _TPW)],
                    out.at[c, pl.ds(s * _TPW, _TPW)])


def _tc_first_body(deg_ref, x_ref, w1_ref, dinv_ref, g_ref):
    deg = jnp.sum(deg_ref[...], axis=0)[0:_N][:, None] + 1.0
    dinv = lax.rsqrt(deg)
    dinv_ref[...] = dinv
    g_ref[...] = jnp.dot(x_ref[...], w1_ref[...],
                         preferred_element_type=jnp.float32) * dinv


def _tc_layer_body(p_ref, dinv_ref, b_ref, w_ref, gout_ref):
    dinv = dinv_ref[...]
    h = jnp.maximum((p_ref[0, 0:_N] + p_ref[1, 0:_N]) * dinv
                    + b_ref[...], 0.0)
    gout_ref[...] = jnp.dot(h, w_ref[...],
                            preferred_element_type=jnp.float32) * dinv


def _tc_final_body(p_ref, dinv_ref, b_ref, batch_ref, wf1_ref,
                   bf1_ref, wf2_ref, bf2_ref, out_ref):
    h = jnp.maximum(
        (p_ref[0, 0:_N] + p_ref[1, 0:_N]) * dinv_ref[...]
        + b_ref[...], 0.0)
    gid = lax.broadcasted_iota(jnp.int32, (_N, _G), 1)
    oh = (batch_ref[...] == gid).astype(jnp.float32)
    sums = lax.dot_general(oh, h, (((0,), (0,)), ((), ())),
                           preferred_element_type=jnp.float32)
    counts = jnp.sum(oh, axis=0)[:, None]
    pooled = sums / jnp.maximum(counts, 1.0)
    z = jnp.maximum(
        jnp.dot(pooled, wf1_ref[...], preferred_element_type=jnp.float32)
        + bf1_ref[...], 0.0)
    logits = jnp.dot(z, wf2_ref[...],
                     preferred_element_type=jnp.float32) + bf2_ref[...]
    m = jnp.max(logits, axis=1, keepdims=True)
    lse = jnp.log(jnp.sum(jnp.exp(logits - m), axis=1, keepdims=True)) + m
    out_ref[...] = logits - lse


_tc_first = pl.pallas_call(
    _tc_first_body,
    out_shape=[jax.ShapeDtypeStruct((_N, 1), jnp.float32),
               jax.ShapeDtypeStruct((_N, _H), jnp.float32)],
)

_tc_layer = pl.pallas_call(
    _tc_layer_body,
    out_shape=jax.ShapeDtypeStruct((_N, _H), jnp.float32),
)

_tc_final = pl.pallas_call(
    _tc_final_body,
    out_shape=jax.ShapeDtypeStruct((_G, _C), jnp.float32),
)


def kernel(x, edge_index, batch, W1, b1, W2, b2, W3, b3, W4, b4,
           Wf1, bf1, Wf2, bf2):
    src_r = edge_index[:, 0].reshape(_NW, _NCH, _CH)
    dst_r = edge_index[:, 1].reshape(_NW, _NCH, _CH)
    batch2 = batch.astype(jnp.int32).reshape(_N, 1)

    zero_h = jnp.zeros((_TPW, _H), jnp.float32)

    deg = _sc_degree(edge_index[:, 1]).reshape(_NW, _NP)
    dinv, g = _tc_first(deg, x, W1)

    for b_prev, w_next in ((b1, W2), (b2, W3), (b3, W4)):
        p = _sc_edge_sum(g, src_r, dst_r, zero_h)
        g = _tc_layer(p, dinv, b_prev.reshape(1, _H), w_next)

    p = _sc_edge_sum(g, src_r, dst_r, zero_h)
    return _tc_final(p, dinv, b4.reshape(1, _H), batch2, Wf1,
                     bf1.reshape(1, _H), Wf2, bf2.reshape(1, _C))


# speedup vs baseline: 1.0062x; 1.0062x over previous
import functools

import jax
import jax.numpy as jnp
from jax import lax
from jax.experimental import pallas as pl
from jax.experimental.pallas import tpu as pltpu
from jax.experimental.pallas import tpu_sc as plsc

_N = 10000
_E = 320000
_H = 128
_G = 64
_C = 3

_NC = 2
_NS = 16
_NW = _NC * _NS
_EPW = _E // _NW
_CH = 125
_NCH = _EPW // _CH
_NBUF = 2
_PH = 16
_TPW = 10240 // _NS
_DW = 128
_NP = 10240
_PAD = _NP - _N

_mesh = plsc.VectorSubcoreMesh(core_axis_name="c", subcore_axis_name="s")


@functools.partial(
    pl.kernel,
    out_type=jax.ShapeDtypeStruct((_NW * _NP,), jnp.float32),
    mesh=_mesh,
    compiler_params=pltpu.CompilerParams(needs_layout_passes=False),
    scratch_types=[
        pltpu.VMEM((_EPW,), jnp.int32),
        pltpu.VMEM((_NP,), jnp.float32),
    ],
)
def _sc_degree(dst_flat, out, dstv, hist):
    c = lax.axis_index("c")
    s = lax.axis_index("s")
    w = s * _NC + c
    pltpu.sync_copy(dst_flat.at[pl.ds(w * _EPW, _EPW)], dstv)
    zero16 = jnp.zeros((16,), jnp.float32)

    def zbody(i, carry):
        hist[pl.ds(i * 16, 16)] = zero16
        return carry

    lax.fori_loop(0, _NP // 16, zbody, 0)
    ones16 = jnp.ones((16,), jnp.float32)

    def body(i, carry):
        for u in range(5):
            idx = dstv[pl.ds((i * 5 + u) * 16, 16)]
            plsc.addupdate_scatter(hist, [idx], ones16)
        return carry

    lax.fori_loop(0, _EPW // 80, body, 0)
    pltpu.sync_copy(hist, out.at[pl.ds(w * _NP, _NP)])


@functools.partial(
    pl.kernel,
    out_type=jax.ShapeDtypeStruct((_NC, _NP, _H), jnp.float32),
    mesh=_mesh,
    scratch_types=[
        pltpu.VMEM((_PH, _CH), jnp.int32) for _ in range(4)
    ] + [pltpu.VMEM((_CH, _H), jnp.float32) for _ in range(_NBUF)]
      + [pltpu.SemaphoreType.DMA for _ in range(_NBUF + 2)]
      + [pltpu.VMEM_SHARED((_NP, _H), jnp.float32)],
)
def _sc_edge_sum(g_hbm, src_r, dst_r, zero_hbm, out,
                 srcv0, srcv1, dstv0, dstv1, *rest):
    c = lax.axis_index("c")
    s = lax.axis_index("s")
    bufs = rest[:_NBUF]
    gsems = rest[_NBUF:2 * _NBUF]
    isems = rest[2 * _NBUF:2 * _NBUF + 2]
    srcvs = (srcv0, srcv1)
    dstvs = (dstv0, dstv1)
    acc = rest[-1]
    w = s * _NC + c

    def _idx_fetch(ph, p):
        pltpu.async_copy(src_r.at[w, pl.ds(ph * _PH, _PH)], srcvs[p],
                         isems[0])
        pltpu.async_copy(dst_r.at[w, pl.ds(ph * _PH, _PH)], dstvs[p],
                         isems[1])

    def _idx_wait(ph, p):
        pltpu.make_async_copy(src_r.at[w, pl.ds(ph * _PH, _PH)], srcvs[p],
                              isems[0]).wait()
        pltpu.make_async_copy(dst_r.at[w, pl.ds(ph * _PH, _PH)], dstvs[p],
                              isems[1]).wait()

    def _gather(srcv, j, b):
        pltpu.async_copy(g_hbm.at[srcv.at[j]], bufs[b], gsems[b])

    def _gwait(srcv, j, b):
        pltpu.make_async_copy(g_hbm.at[srcv.at[j]], bufs[b], gsems[b]).wait()

    def _scatter(dstv, j, b):
        pltpu.sync_copy(bufs[b], acc.at[dstv.at[j]], add=True)

    _idx_fetch(0, 0)
    _last = _N - (_NS - 1) * _TPW

    @pl.when((c == 0) & (s < _NS - 1))
    def _():
        pltpu.sync_copy(g_hbm.at[pl.ds(s * _TPW, _TPW)],
                        acc.at[pl.ds(s * _TPW, _TPW)])

    @pl.when((c == 0) & (s == _NS - 1))
    def _():
        pltpu.sync_copy(g_hbm.at[pl.ds(_N - _last, _last)],
                        acc.at[pl.ds(_N - _last, _last)])

    @pl.when((c == 1) & (s < _NS - 1))
    def _():
        pltpu.sync_copy(zero_hbm, acc.at[pl.ds(s * _TPW, _TPW)])

    @pl.when((c == 1) & (s == _NS - 1))
    def _():
        pltpu.sync_copy(zero_hbm.at[pl.ds(0, _last)],
                        acc.at[pl.ds(_N - _last, _last)])

    _nph = _NCH // _PH
    for ph in range(_nph):
        p = ph % 2
        _idx_wait(ph, p)
        if ph + 1 < _nph:
            _idx_fetch(ph + 1, 1 - p)
        srcv, dstv = srcvs[p], dstvs[p]
        for b in range(_NBUF):
            _gather(srcv, b, b)
        if ph == 0:
            plsc.subcore_barrier()

        def body(i, carry, srcv=srcv, dstv=dstv):
            j0 = i * _NBUF
            for b in range(_NBUF):
                j = j0 + b
                _gwait(srcv, j, b)
                _scatter(dstv, j, b)

                @pl.when(j + _NBUF < _PH)
                def _():
                    _gather(srcv, j + _NBUF, b)
            return carry

        lax.fori_loop(0, _PH // _NBUF, body, 0)
    plsc.subcore_barrier()

    @pl.when(s < _NS - 1)
    def _():
        pltpu.sync_copy(acc.at[pl.ds(s * _TPW, _TPW)],
                        out.at[c, pl.ds(s * _TPW, _TPW)])

    @pl.when(s == _NS - 1)
    def _():
        pltpu.sync_copy(acc.at[pl.ds(_N - _last, _last)],
                        out.at[c, pl.ds(_N - _last, _last)])


def _tc_first_body(deg_ref, x_ref, w1_ref, dinv_ref, g_ref):
    deg = jnp.sum(deg_ref[...], axis=0)[0:_N][:, None] + 1.0
    dinv = lax.rsqrt(deg)
    dinv_ref[...] = dinv
    g_ref[...] = jnp.dot(x_ref[...], w1_ref[...],
                         preferred_element_type=jnp.float32) * dinv


def _tc_layer_body(p_ref, dinv_ref, b_ref, w_ref, gout_ref):
    dinv = dinv_ref[...]
    h = jnp.maximum((p_ref[0, 0:_N] + p_ref[1, 0:_N]) * dinv
                    + b_ref[...], 0.0)
    gout_ref[...] = jnp.dot(h, w_ref[...],
                            preferred_element_type=jnp.float32) * dinv


def _tc_final_body(p_ref, dinv_ref, b_ref, batch_ref, wf1_ref,
                   bf1_ref, wf2_ref, bf2_ref, out_ref):
    h = jnp.maximum(
        (p_ref[0, 0:_N] + p_ref[1, 0:_N]) * dinv_ref[...]
        + b_ref[...], 0.0)
    gid = lax.broadcasted_iota(jnp.int32, (_N, _G), 1)
    oh = (batch_ref[...] == gid).astype(jnp.float32)
    sums = lax.dot_general(oh, h, (((0,), (0,)), ((), ())),
                           preferred_element_type=jnp.float32)
    counts = jnp.sum(oh, axis=0)[:, None]
    pooled = sums / jnp.maximum(counts, 1.0)
    z = jnp.maximum(
        jnp.dot(pooled, wf1_ref[...], preferred_element_type=jnp.float32)
        + bf1_ref[...], 0.0)
    logits = jnp.dot(z, wf2_ref[...],
                     preferred_element_type=jnp.float32) + bf2_ref[...]
    m = jnp.max(logits, axis=1, keepdims=True)
    lse = jnp.log(jnp.sum(jnp.exp(logits - m), axis=1, keepdims=True)) + m
    out_ref[...] = logits - lse


_tc_first = pl.pallas_call(
    _tc_first_body,
    out_shape=[jax.ShapeDtypeStruct((_N, 1), jnp.float32),
               jax.ShapeDtypeStruct((_N, _H), jnp.float32)],
)

_tc_layer = pl.pallas_call(
    _tc_layer_body,
    out_shape=jax.ShapeDtypeStruct((_N, _H), jnp.float32),
)

_tc_final = pl.pallas_call(
    _tc_final_body,
    out_shape=jax.ShapeDtypeStruct((_G, _C), jnp.float32),
)


def kernel(x, edge_index, batch, W1, b1, W2, b2, W3, b3, W4, b4,
           Wf1, bf1, Wf2, bf2):
    src_r = edge_index[:, 0].reshape(_NW, _NCH, _CH)
    dst_r = edge_index[:, 1].reshape(_NW, _NCH, _CH)
    batch2 = batch.astype(jnp.int32).reshape(_N, 1)

    zero_h = jnp.zeros((_TPW, _H), jnp.float32)

    deg = _sc_degree(edge_index[:, 1]).reshape(_NW, _NP)
    dinv, g = _tc_first(deg, x, W1)

    for b_prev, w_next in ((b1, W2), (b2, W3), (b3, W4)):
        p = _sc_edge_sum(g, src_r, dst_r, zero_h)
        g = _tc_layer(p, dinv, b_prev.reshape(1, _H), w_next)

    p = _sc_edge_sum(g, src_r, dst_r, zero_h)
    return _tc_final(p, dinv, b4.reshape(1, _H), batch2, Wf1,
                     bf1.reshape(1, _H), Wf2, bf2.reshape(1, _C))


# speedup vs baseline: 1.0288x; 1.0224x over previous
import functools

import jax
import jax.numpy as jnp
from jax import lax
from jax.experimental import pallas as pl
from jax.experimental.pallas import tpu as pltpu
from jax.experimental.pallas import tpu_sc as plsc

_N = 10000
_E = 320000
_H = 128
_G = 64
_C = 3

_NC = 2
_NS = 16
_NW = _NC * _NS
_EPW = _E // _NW
_CH = 125
_NCH = _EPW // _CH
_NBUF = 2
_PH = 40
_TPW = 10240 // _NS
_DW = 128
_NP = 10240
_PAD = _NP - _N

_mesh = plsc.VectorSubcoreMesh(core_axis_name="c", subcore_axis_name="s")


@functools.partial(
    pl.kernel,
    out_type=jax.ShapeDtypeStruct((_NW * _NP,), jnp.float32),
    mesh=_mesh,
    compiler_params=pltpu.CompilerParams(needs_layout_passes=False),
    scratch_types=[
        pltpu.VMEM((_EPW,), jnp.int32),
        pltpu.VMEM((_NP,), jnp.float32),
    ],
)
def _sc_degree(dst_flat, out, dstv, hist):
    c = lax.axis_index("c")
    s = lax.axis_index("s")
    w = s * _NC + c
    pltpu.sync_copy(dst_flat.at[pl.ds(w * _EPW, _EPW)], dstv)
    zero16 = jnp.zeros((16,), jnp.float32)

    def zbody(i, carry):
        hist[pl.ds(i * 16, 16)] = zero16
        return carry

    lax.fori_loop(0, _NP // 16, zbody, 0)
    ones16 = jnp.ones((16,), jnp.float32)

    def body(i, carry):
        for u in range(5):
            idx = dstv[pl.ds((i * 5 + u) * 16, 16)]
            plsc.addupdate_scatter(hist, [idx], ones16)
        return carry

    lax.fori_loop(0, _EPW // 80, body, 0)
    pltpu.sync_copy(hist, out.at[pl.ds(w * _NP, _NP)])


@functools.partial(
    pl.kernel,
    out_type=jax.ShapeDtypeStruct((_NC, _NP, _H), jnp.float32),
    mesh=_mesh,
    scratch_types=[
        pltpu.VMEM((_PH, _CH), jnp.int32) for _ in range(2)
    ] + [pltpu.VMEM((_CH, _H), jnp.float32) for _ in range(_NBUF)]
      + [pltpu.SemaphoreType.DMA for _ in range(_NBUF + 2)]
      + [pltpu.VMEM_SHARED((_NP, _H), jnp.float32)],
)
def _sc_edge_sum(g_hbm, src_r, dst_r, zero_hbm, out, srcv0, dstv0, *rest):
    c = lax.axis_index("c")
    s = lax.axis_index("s")
    bufs = rest[:_NBUF]
    gsems = rest[_NBUF:2 * _NBUF]
    isems = rest[2 * _NBUF:2 * _NBUF + 2]
    acc = rest[-1]
    w = s * _NC + c

    def _idx_fetch(ph):
        pltpu.async_copy(src_r.at[w, pl.ds(ph * _PH, _PH)], srcv0, isems[0])
        pltpu.async_copy(dst_r.at[w, pl.ds(ph * _PH, _PH)], dstv0, isems[1])

    def _idx_wait(ph):
        pltpu.make_async_copy(src_r.at[w, pl.ds(ph * _PH, _PH)], srcv0,
                              isems[0]).wait()
        pltpu.make_async_copy(dst_r.at[w, pl.ds(ph * _PH, _PH)], dstv0,
                              isems[1]).wait()

    def _gather(srcv, j, b):
        pltpu.async_copy(g_hbm.at[srcv.at[j]], bufs[b], gsems[b])

    def _gwait(srcv, j, b):
        pltpu.make_async_copy(g_hbm.at[srcv.at[j]], bufs[b], gsems[b]).wait()

    def _scatter(dstv, j, b):
        pltpu.sync_copy(bufs[b], acc.at[dstv.at[j]], add=True)

    _idx_fetch(0)
    _last = _N - (_NS - 1) * _TPW

    @pl.when((c == 0) & (s < _NS - 1))
    def _():
        pltpu.sync_copy(g_hbm.at[pl.ds(s * _TPW, _TPW)],
                        acc.at[pl.ds(s * _TPW, _TPW)])

    @pl.when((c == 0) & (s == _NS - 1))
    def _():
        pltpu.sync_copy(g_hbm.at[pl.ds(_N - _last, _last)],
                        acc.at[pl.ds(_N - _last, _last)])

    @pl.when((c == 1) & (s < _NS - 1))
    def _():
        pltpu.sync_copy(zero_hbm, acc.at[pl.ds(s * _TPW, _TPW)])

    @pl.when((c == 1) & (s == _NS - 1))
    def _():
        pltpu.sync_copy(zero_hbm.at[pl.ds(0, _last)],
                        acc.at[pl.ds(_N - _last, _last)])

    _nph = _NCH // _PH
    for ph in range(_nph):
        _idx_wait(ph)
        srcv, dstv = srcv0, dstv0
        for b in range(_NBUF):
            _gather(srcv, b, b)
        if ph == 0:
            plsc.subcore_barrier()

        def body(i, carry, srcv=srcv, dstv=dstv):
            j0 = i * _NBUF
            for b in range(_NBUF):
                j = j0 + b
                _gwait(srcv, j, b)
                _scatter(dstv, j, b)

                @pl.when(j + _NBUF < _PH)
                def _():
                    _gather(srcv, j + _NBUF, b)
            return carry

        lax.fori_loop(0, _PH // _NBUF, body, 0)
        if ph + 1 < _nph:
            _idx_fetch(ph + 1)
    plsc.subcore_barrier()

    @pl.when(s < _NS - 1)
    def _():
        pltpu.sync_copy(acc.at[pl.ds(s * _TPW, _TPW)],
                        out.at[c, pl.ds(s * _TPW, _TPW)])

    @pl.when(s == _NS - 1)
    def _():
        pltpu.sync_copy(acc.at[pl.ds(_N - _last, _last)],
                        out.at[c, pl.ds(_N - _last, _last)])


def _tc_first_body(deg_ref, x_ref, w1_ref, dinv_ref, g_ref):
    deg = jnp.sum(deg_ref[...], axis=0)[0:_N][:, None] + 1.0
    dinv = lax.rsqrt(deg)
    dinv_ref[...] = dinv
    g_ref[...] = jnp.dot(x_ref[...], w1_ref[...],
                         preferred_element_type=jnp.float32) * dinv


def _tc_layer_body(p_ref, dinv_ref, b_ref, w_ref, gout_ref):
    dinv = dinv_ref[...]
    h = jnp.maximum((p_ref[0, 0:_N] + p_ref[1, 0:_N]) * dinv
                    + b_ref[...], 0.0)
    gout_ref[...] = jnp.dot(h, w_ref[...],
                            preferred_element_type=jnp.float32) * dinv


def _tc_final_body(p_ref, dinv_ref, b_ref, batch_ref, wf1_ref,
                   bf1_ref, wf2_ref, bf2_ref, out_ref):
    h = jnp.maximum(
        (p_ref[0, 0:_N] + p_ref[1, 0:_N]) * dinv_ref[...]
        + b_ref[...], 0.0)
    gid = lax.broadcasted_iota(jnp.int32, (_N, _G), 1)
    oh = (batch_ref[...] == gid).astype(jnp.float32)
    sums = lax.dot_general(oh, h, (((0,), (0,)), ((), ())),
                           preferred_element_type=jnp.float32)
    counts = jnp.sum(oh, axis=0)[:, None]
    pooled = sums / jnp.maximum(counts, 1.0)
    z = jnp.maximum(
        jnp.dot(pooled, wf1_ref[...], preferred_element_type=jnp.float32)
        + bf1_ref[...], 0.0)
    logits = jnp.dot(z, wf2_ref[...],
                     preferred_element_type=jnp.float32) + bf2_ref[...]
    m = jnp.max(logits, axis=1, keepdims=True)
    lse = jnp.log(jnp.sum(jnp.exp(logits - m), axis=1, keepdims=True)) + m
    out_ref[...] = logits - lse


_tc_first = pl.pallas_call(
    _tc_first_body,
    out_shape=[jax.ShapeDtypeStruct((_N, 1), jnp.float32),
               jax.ShapeDtypeStruct((_N, _H), jnp.float32)],
)

_tc_layer = pl.pallas_call(
    _tc_layer_body,
    out_shape=jax.ShapeDtypeStruct((_N, _H), jnp.float32),
)

_tc_final = pl.pallas_call(
    _tc_final_body,
    out_shape=jax.ShapeDtypeStruct((_G, _C), jnp.float32),
)


def kernel(x, edge_index, batch, W1, b1, W2, b2, W3, b3, W4, b4,
           Wf1, bf1, Wf2, bf2):
    src_r = edge_index[:, 0].reshape(_NW, _NCH, _CH)
    dst_r = edge_index[:, 1].reshape(_NW, _NCH, _CH)
    batch2 = batch.astype(jnp.int32).reshape(_N, 1)

    zero_h = jnp.zeros((_TPW, _H), jnp.float32)

    deg = _sc_degree(edge_index[:, 1]).reshape(_NW, _NP)
    dinv, g = _tc_first(deg, x, W1)

    for b_prev, w_next in ((b1, W2), (b2, W3), (b3, W4)):
        p = _sc_edge_sum(g, src_r, dst_r, zero_h)
        g = _tc_layer(p, dinv, b_prev.reshape(1, _H), w_next)

    p = _sc_edge_sum(g, src_r, dst_r, zero_h)
    return _tc_final(p, dinv, b4.reshape(1, _H), batch2, Wf1,
                     bf1.reshape(1, _H), Wf2, bf2.reshape(1, _C))
